# Initial kernel scaffold; baseline (speedup 1.0000x reference)
import jax
import jax.numpy as jnp
from jax import lax
from jax.experimental import pallas as pl
from jax.experimental.pallas import tpu as pltpu
from jax.experimental.pallas import tpu_sc as plsc

_N_EMB = 26
_VOCAB = 100000
_EMB_DIM = 32
_B = 1024
_L = 50
_N_NUM = 2

_BL = _B * _L
_EMB_COLS = _N_EMB * _EMB_DIM
_D_OUT = _EMB_COLS + _N_NUM

_NC = 2
_NS = 16
_NW = _NC * _NS
_PER_W = _BL // _NW
_CH = 64
_CHUNKS = _PER_W // _CH


def _body(idx_hbm, num_hbm, tab_hbm, out_hbm, idx_v, num_v, row_v, sem):
    wid = lax.axis_index("s") * _NC + lax.axis_index("c")
    lane = jax.lax.iota(jnp.int32, 16)

    @pl.loop(0, _CHUNKS)
    def _chunk(g):
        base = wid * _PER_W + g * _CH
        pltpu.sync_copy(idx_hbm.at[:, pl.ds(base, _CH)], idx_v)
        pltpu.sync_copy(num_hbm.at[:, pl.ds(base, _CH)], num_v)

        for e in range(1, _N_EMB):
            for j in range(_CH // 16):
                sl = pl.ds(j * 16, 16)
                idx_v[e, sl] = idx_v[e, sl] + jnp.full((16,), e * _VOCAB, jnp.int32)

        descs = [
            pltpu.async_copy(
                tab_hbm.at[idx_v.at[e]],
                row_v.at[:, pl.ds(e * _EMB_DIM, _EMB_DIM)],
                sem,
            )
            for e in range(_N_EMB)
        ]

        for n in range(_N_NUM):
            col = jnp.full((16,), _EMB_COLS + n, jnp.int32)
            for j in range(_CH // 16):
                v = num_v[n, pl.ds(j * 16, 16)]
                plsc.store_scatter(row_v, [lane + j * 16, col], v)

        for d in descs:
            d.wait()
        pltpu.sync_copy(row_v, out_hbm.at[pl.ds(base, _CH)])


@jax.jit
def _sc_lookup(idx2, num2, tab2):
    mesh = plsc.VectorSubcoreMesh(core_axis_name="c", subcore_axis_name="s")
    return pl.kernel(
        _body,
        out_type=jax.ShapeDtypeStruct((_BL, _D_OUT), jnp.float32),
        mesh=mesh,
        scratch_types=[
            pltpu.VMEM((_N_EMB, _CH), jnp.int32),
            pltpu.VMEM((_N_NUM, _CH), jnp.float32),
            pltpu.VMEM((_CH, _D_OUT), jnp.float32),
            pltpu.SemaphoreType.DMA,
        ],
    )(idx2, num2, tab2)


def kernel(idx, numeric_feats, event_time, tables):
    idx2 = idx.reshape(_N_EMB, _BL).astype(jnp.int32)
    num2 = numeric_feats.reshape(_N_NUM, _BL).astype(jnp.float32)
    tab2 = tables.reshape(_N_EMB * _VOCAB, _EMB_DIM)
    out = _sc_lookup(idx2, num2, tab2)
    return (out.reshape(_B, _L, _D_OUT), event_time)


# speedup vs baseline: 3.2080x; 3.2080x over previous
import jax
import jax.numpy as jnp
from jax import lax
from jax.experimental import pallas as pl
from jax.experimental.pallas import tpu as pltpu
from jax.experimental.pallas import tpu_sc as plsc

_N_EMB = 26
_VOCAB = 100000
_EMB_DIM = 32
_B = 1024
_L = 50
_N_NUM = 2

_BL = _B * _L
_EMB_COLS = _N_EMB * _EMB_DIM
_D_OUT = _EMB_COLS + _N_NUM

_NC = 2
_NS = 16
_NW = _NC * _NS
_PER_W = _BL // _NW
_CH = 80
_CHUNKS = _PER_W // _CH


def _body(idx_hbm, num_hbm, tab_hbm, out_hbm, idx_v, num_v, g3, np_v, sem, wsem):
    wid = lax.axis_index("s") * _NC + lax.axis_index("c")
    span = wid * _PER_W
    lane = jax.lax.iota(jnp.int32, 16)

    for e in range(_N_EMB):
        pltpu.sync_copy(idx_hbm.at[pl.ds(e * _BL + span, _PER_W)], idx_v.at[e])
    for n in range(_N_NUM):
        pltpu.sync_copy(num_hbm.at[pl.ds(n * _BL + span, _PER_W)], num_v.at[n])

    for e in range(1, _N_EMB):
        off = jnp.full((16,), e * _VOCAB, jnp.int32)

        @pl.loop(0, _PER_W // 16)
        def _add(j, e=e, off=off):
            sl = pl.ds(j * 16, 16)
            idx_v[e, sl] = idx_v[e, sl] + off

    @pl.loop(0, _CHUNKS)
    def _chunk(g):
        base = g * _CH
        descs = [
            pltpu.async_copy(
                tab_hbm.at[idx_v.at[e, pl.ds(base, _CH)]],
                g3.at[e],
                sem,
            )
            for e in range(_N_EMB)
        ]

        for n in range(_N_NUM):
            col = jnp.full((16,), n, jnp.int32)
            for j in range(_CH // 16):
                v = num_v[n, pl.ds(base + j * 16, 16)]
                plsc.store_scatter(np_v, [lane + j * 16, col], v)

        for d in descs:
            d.wait()
        rows = pl.ds(span + base, _CH)
        writes = [
            pltpu.async_copy(
                g3.at[e],
                out_hbm.at[rows, pl.ds(e * _EMB_DIM, _EMB_DIM)],
                wsem,
            )
            for e in range(_N_EMB)
        ]
        writes.append(
            pltpu.async_copy(
                np_v, out_hbm.at[rows, pl.ds(_EMB_COLS, _N_NUM)], wsem
            )
        )
        for w in writes:
            w.wait()


@jax.jit
def _sc_lookup(idx2, num2, tab2):
    mesh = plsc.VectorSubcoreMesh(core_axis_name="c", subcore_axis_name="s")
    return pl.kernel(
        _body,
        out_type=jax.ShapeDtypeStruct((_BL, _D_OUT), jnp.float32),
        mesh=mesh,
        scratch_types=[
            pltpu.VMEM((_N_EMB, _PER_W), jnp.int32),
            pltpu.VMEM((_N_NUM, _PER_W), jnp.float32),
            pltpu.VMEM((_N_EMB, _CH, _EMB_DIM), jnp.float32),
            pltpu.VMEM((_CH, _N_NUM), jnp.float32),
            pltpu.SemaphoreType.DMA,
            pltpu.SemaphoreType.DMA,
        ],
        compiler_params=pltpu.CompilerParams(
            use_tc_tiling_on_sc=False, needs_layout_passes=False
        ),
    )(idx2, num2, tab2)


def kernel(idx, numeric_feats, event_time, tables):
    idx2 = idx.reshape(_N_EMB * _BL).astype(jnp.int32)
    num2 = numeric_feats.reshape(_N_NUM * _BL).astype(jnp.float32)
    tab2 = tables.reshape(_N_EMB * _VOCAB, _EMB_DIM)
    out = _sc_lookup(idx2, num2, tab2)
    return (out.reshape(_B, _L, _D_OUT), event_time)


# speedup vs baseline: 3.2585x; 1.0157x over previous
import jax
import jax.numpy as jnp
from jax import lax
from jax.experimental import pallas as pl
from jax.experimental.pallas import tpu as pltpu
from jax.experimental.pallas import tpu_sc as plsc

_N_EMB = 26
_VOCAB = 100000
_EMB_DIM = 32
_B = 1024
_L = 50
_N_NUM = 2

_EMB_COLS = _N_EMB * _EMB_DIM
_D_OUT = _EMB_COLS + _N_NUM

_NC = 2
_NS = 16
_NW = _NC * _NS
_B_PER_W = _B // _NW


def _body(idx_hbm, num_hbm, tab_hbm, out_hbm, idx_v, num_v, np_v, g3, sem, wsem):
    wid = lax.axis_index("s") * _NC + lax.axis_index("c")
    b0 = wid * _B_PER_W
    lane = jax.lax.iota(jnp.int32, 16)

    for e in range(_N_EMB):
        pltpu.sync_copy(idx_hbm.at[e, pl.ds(b0, _B_PER_W)], idx_v.at[e])
    for n in range(_N_NUM):
        pltpu.sync_copy(num_hbm.at[n, pl.ds(b0, _B_PER_W)], num_v.at[n])

    for e in range(1, _N_EMB):
        off = jnp.full((16,), e * _VOCAB, jnp.int32)

        @pl.loop(0, _B_PER_W)
        def _add(bb, e=e, off=off):
            for j in range(_L // 16):
                sl = pl.ds(j * 16, 16)
                idx_v[e, bb, sl] = idx_v[e, bb, sl] + off
            tl = pl.ds(_L - 16, 16)
            tmask = lane >= 16 - (_L % 16)
            v = plsc.load_gather(idx_v, [jnp.full((16,), e, jnp.int32),
                                         jnp.full((16,), bb, jnp.int32),
                                         lane + (_L - 16)], mask=tmask)
            plsc.store_scatter(idx_v, [jnp.full((16,), e, jnp.int32),
                                       jnp.full((16,), bb, jnp.int32),
                                       lane + (_L - 16)], v + off, mask=tmask)

    @pl.loop(0, _B_PER_W)
    def _nt(bb):
        bcol = jnp.full((16,), 0, jnp.int32) + bb
        for n in range(_N_NUM):
            ncol = jnp.full((16,), n, jnp.int32)
            for j in range(_L // 16):
                v = num_v[n, bb, pl.ds(j * 16, 16)]
                plsc.store_scatter(np_v, [bcol, lane + j * 16, ncol], v)
        tmask = lane < 2 * (_L % 16)
        pidx = _L - (_L % 16) + (lane >> 1)
        nidx = lane & 1
        v = plsc.load_gather(num_v, [nidx, bcol, pidx], mask=tmask)
        plsc.store_scatter(np_v, [bcol, pidx, nidx], v, mask=tmask)

    @pl.loop(0, _B_PER_W)
    def _row(bb):
        b = b0 + bb
        descs = [
            pltpu.async_copy(
                tab_hbm.at[idx_v.at[e, bb]],
                g3.at[e],
                sem,
            )
            for e in range(_N_EMB)
        ]
        for d in descs:
            d.wait()
        writes = [
            pltpu.async_copy(
                g3.at[e],
                out_hbm.at[b, :, pl.ds(e * _EMB_DIM, _EMB_DIM)],
                wsem,
            )
            for e in range(_N_EMB)
        ]
        writes.append(
            pltpu.async_copy(
                np_v.at[bb], out_hbm.at[b, :, pl.ds(_EMB_COLS, _N_NUM)], wsem
            )
        )
        for w in writes:
            w.wait()


@jax.jit
def _sc_lookup(idx3, num3, tab2):
    mesh = plsc.VectorSubcoreMesh(core_axis_name="c", subcore_axis_name="s")
    return pl.kernel(
        _body,
        out_type=jax.ShapeDtypeStruct((_B, _L, _D_OUT), jnp.float32),
        mesh=mesh,
        scratch_types=[
            pltpu.VMEM((_N_EMB, _B_PER_W, _L), jnp.int32),
            pltpu.VMEM((_N_NUM, _B_PER_W, _L), jnp.float32),
            pltpu.VMEM((_B_PER_W, _L, _N_NUM), jnp.float32),
            pltpu.VMEM((_N_EMB, _L, _EMB_DIM), jnp.float32),
            pltpu.SemaphoreType.DMA,
            pltpu.SemaphoreType.DMA,
        ],
        compiler_params=pltpu.CompilerParams(
            use_tc_tiling_on_sc=False, needs_layout_passes=False
        ),
    )(idx3, num3, tab2)


def kernel(idx, numeric_feats, event_time, tables):
    idx3 = idx.astype(jnp.int32)
    num3 = numeric_feats.astype(jnp.float32)
    tab2 = tables.reshape(_N_EMB * _VOCAB, _EMB_DIM)
    out = _sc_lookup(idx3, num3, tab2)
    return (out, event_time)
